# baseline (speedup 1.0000x reference)
import functools

import jax
import jax.numpy as jnp
from jax import lax
from jax.experimental import pallas as pl
from jax.experimental.pallas import tpu as pltpu
from jax.experimental.pallas import tpu_sc as plsc

NODES = 10000
PAD = 10240
EDGES = 320000
D = 128
NC = 2
NS = 16
NW = NC * NS
EPW = EDGES // NW
K = 80
CHUNKS = EPW // K
ZR = 128
RPT = PAD // NS
BR = 1024


CROWS = EDGES // K
RPTILE = 128
CROWS_PAD = NW * RPTILE
RING = 4


def _make_sc_agg():
    mesh = plsc.VectorSubcoreMesh(core_axis_name="c", subcore_axis_name="s")
    out_type = jax.ShapeDtypeStruct((NC, PAD, D), jnp.float32)
    scratch = (
        [pltpu.VMEM((1, 2, K), jnp.int32) for _ in range(RING)]
        + [pltpu.VMEM((K, D), jnp.float32) for _ in range(RING)]
        + [pltpu.VMEM_SHARED((PAD, D), jnp.float32)]
        + [pltpu.SemaphoreType.DMA] * (2 * RING)
    )

    def body(x_hbm, sd_hbm, out_hbm, *rest):
        idx = rest[0:RING]
        rows = rest[RING:2 * RING]
        acc = rest[2 * RING]
        gsem = rest[2 * RING + 1:2 * RING + 1 + RING]
        ssem = rest[2 * RING + 1 + RING:]
        c = lax.axis_index("c")
        s = lax.axis_index("s")
        wid = s * NC + c
        zero16 = jnp.zeros((16,), jnp.float32)

        def zero_rows0(i, carry):
            for j in range(D // 16):
                rows[0][i, pl.ds(j * 16, 16)] = zero16
            return carry

        lax.fori_loop(0, K, zero_rows0, 0)
        r0 = s * RPT
        for kk in range(RPT // K):
            pltpu.sync_copy(rows[0], acc.at[pl.ds(r0 + kk * K, K)])
        plsc.subcore_barrier()

        base = wid * RPTILE

        def drain(sem, b):
            pltpu.make_async_copy(x_hbm.at[pl.ds(0, K)], rows[b], sem).wait()

        def fire_gather(b):
            pltpu.async_copy(x_hbm.at[idx[b].at[0, 0]], rows[b], gsem[b])

        def step(t, b, bn, do_drain):
            if do_drain:
                drain(ssem[bn], bn)
            nxt = pltpu.async_copy(sd_hbm.at[pl.ds(base + t + 1, 1)],
                                   idx[bn], gsem[bn])
            drain_g = pltpu.make_async_copy(x_hbm.at[pl.ds(0, K)], rows[b],
                                            gsem[b])
            drain_g.wait()
            pltpu.async_copy(rows[b], acc.at[idx[b].at[0, 1]], ssem[b],
                             add=True)
            nxt.wait()
            fire_gather(bn)

        pltpu.sync_copy(sd_hbm.at[pl.ds(base, 1)], idx[0])
        fire_gather(0)
        step(0, 0, 1, False)
        step(1, 1, 2, False)
        step(2, 2, 3, False)
        step(3, 3, 0, True)

        def quad(i, carry):
            t = 4 + 4 * i
            step(t, 0, 1, True)
            step(t + 1, 1, 2, True)
            step(t + 2, 2, 3, True)
            step(t + 3, 3, 0, True)
            return carry

        lax.fori_loop(0, (RPTILE - 4) // 4, quad, 0)
        drain(ssem[1], 1)
        drain(ssem[2], 2)
        drain(ssem[3], 3)
        drain(gsem[0], 0)
        plsc.subcore_barrier()
        pltpu.sync_copy(acc.at[pl.ds(s * RPT, RPT)],
                        out_hbm.at[c, pl.ds(s * RPT, RPT)])

    return functools.partial(
        pl.kernel, mesh=mesh, out_type=out_type,
        scratch_types=tuple(scratch),
        compiler_params=pltpu.CompilerParams(needs_layout_passes=False))(body)


def _make_sc_deg():
    mesh = plsc.VectorSubcoreMesh(core_axis_name="c", subcore_axis_name="s")
    out_type = jax.ShapeDtypeStruct((NW, PAD), jnp.float32)
    scratch = [
        pltpu.VMEM((RPTILE, 2, K), jnp.int32),
        pltpu.VMEM((PAD,), jnp.float32),
    ]

    def body(sd_hbm, degp_hbm, idxall, deg_v):
        c = lax.axis_index("c")
        s = lax.axis_index("s")
        wid = s * NC + c
        zero16 = jnp.zeros((16,), jnp.float32)
        ones16 = jnp.full((16,), 1.0, jnp.float32)

        def zero_deg(i, carry):
            deg_v[pl.ds(i * 16, 16)] = zero16
            return carry

        lax.fori_loop(0, PAD // 16, zero_deg, 0)
        pltpu.sync_copy(sd_hbm.at[pl.ds(wid * RPTILE, RPTILE)], idxall)

        def row(r, carry):
            for q in range(K // 16):
                idx = idxall[r, 1, pl.ds(q * 16, 16)]
                plsc.addupdate_scatter(deg_v, [idx], ones16)
            return carry

        lax.fori_loop(0, RPTILE, row, 0)
        pltpu.sync_copy(deg_v, degp_hbm.at[wid])

    return functools.partial(
        pl.kernel, mesh=mesh, out_type=out_type,
        scratch_types=tuple(scratch),
        compiler_params=pltpu.CompilerParams(needs_layout_passes=False))(body)


def _mm_body(x_ref, w_ref, o_ref):
    o_ref[...] = jnp.dot(x_ref[...], w_ref[...],
                         preferred_element_type=jnp.float32)


def _mm(x, w):
    return pl.pallas_call(
        _mm_body,
        grid=(PAD // BR,),
        in_specs=[pl.BlockSpec((BR, D), lambda i: (i, 0)),
                  pl.BlockSpec((D, D), lambda i: (0, 0))],
        out_specs=pl.BlockSpec((BR, D), lambda i: (i, 0)),
        out_shape=jax.ShapeDtypeStruct((PAD, D), jnp.float32),
    )(x, w)


def _combine_body(h_ref, p_ref, degt_ref, ws_ref, b_ref, wn_ref,
                  o1_ref, o2_ref):
    deg = jnp.sum(degt_ref[...], axis=1, keepdims=True)
    inv = 1.0 / jnp.maximum(deg, 1.0)
    agg = (p_ref[0] + p_ref[1]) * inv
    t = jnp.dot(h_ref[...], ws_ref[...],
                preferred_element_type=jnp.float32) + b_ref[...] + agg
    hr = jnp.maximum(t, 0.0)
    o1_ref[...] = hr
    o2_ref[...] = jnp.dot(hr, wn_ref[...],
                          preferred_element_type=jnp.float32)


def _combine(h, p, degt, ws, b, wn):
    return pl.pallas_call(
        _combine_body,
        grid=(PAD // BR,),
        in_specs=[pl.BlockSpec((BR, D), lambda i: (i, 0)),
                  pl.BlockSpec((NC, BR, D), lambda i: (0, i, 0)),
                  pl.BlockSpec((BR, NW), lambda i: (i, 0)),
                  pl.BlockSpec((D, D), lambda i: (0, 0)),
                  pl.BlockSpec((1, D), lambda i: (0, 0)),
                  pl.BlockSpec((D, D), lambda i: (0, 0))],
        out_specs=[pl.BlockSpec((BR, D), lambda i: (i, 0)),
                   pl.BlockSpec((BR, D), lambda i: (i, 0))],
        out_shape=[jax.ShapeDtypeStruct((PAD, D), jnp.float32),
                   jax.ShapeDtypeStruct((PAD, D), jnp.float32)],
    )(h, p, degt, ws, b, wn)


def _final_body(h_ref, p_ref, degt_ref, ws_ref, b_ref, o_ref):
    deg = jnp.sum(degt_ref[...], axis=1, keepdims=True)
    inv = 1.0 / jnp.maximum(deg, 1.0)
    agg = (p_ref[0] + p_ref[1]) * inv
    o_ref[...] = jnp.dot(h_ref[...], ws_ref[...],
                         preferred_element_type=jnp.float32) + b_ref[...] + agg


def _final(h, p, degt, ws, b):
    return pl.pallas_call(
        _final_body,
        grid=(PAD // BR,),
        in_specs=[pl.BlockSpec((BR, D), lambda i: (i, 0)),
                  pl.BlockSpec((NC, BR, D), lambda i: (0, i, 0)),
                  pl.BlockSpec((BR, NW), lambda i: (i, 0)),
                  pl.BlockSpec((D, D), lambda i: (0, 0)),
                  pl.BlockSpec((1, D), lambda i: (0, 0))],
        out_specs=pl.BlockSpec((BR, D), lambda i: (i, 0)),
        out_shape=jax.ShapeDtypeStruct((PAD, D), jnp.float32),
    )(h, p, degt, ws, b)


def kernel(h, edge_index, W_self0, W_neigh0, b0, W_self1, W_neigh1, b1,
           W_self2, W_neigh2, b2):
    sd = jnp.stack([edge_index[0].reshape(CROWS, K),
                    edge_index[1].reshape(CROWS, K)], axis=1)
    npad = CROWS_PAD - CROWS + 1
    trash = NODES + (jnp.arange(npad * K, dtype=jnp.int32) % (PAD - NODES))
    trash = trash.reshape(npad, 1, K)
    sd = jnp.concatenate(
        [sd, jnp.broadcast_to(trash, (npad, 2, K))], axis=0)
    h_pad = jnp.pad(h, ((0, PAD - NODES), (0, 0)))
    b0r = b0.reshape(1, D)
    b1r = b1.reshape(1, D)
    b2r = b2.reshape(1, D)

    sc_agg = _make_sc_agg()
    sc_deg = _make_sc_deg()

    degp = sc_deg(sd)
    hn0 = _mm(h_pad, W_neigh0)
    p0 = sc_agg(hn0, sd)
    degt = degp.T
    h1, hn1 = _combine(h_pad, p0, degt, W_self0, b0r, W_neigh1)
    p1 = sc_agg(hn1, sd)
    h2, hn2 = _combine(h1, p1, degt, W_self1, b1r, W_neigh2)
    p2 = sc_agg(hn2, sd)
    out = _final(h2, p2, degt, W_self2, b2r)
    return out[:NODES]


# speedup vs baseline: 1.0679x; 1.0679x over previous
import functools

import jax
import jax.numpy as jnp
from jax import lax
from jax.experimental import pallas as pl
from jax.experimental.pallas import tpu as pltpu
from jax.experimental.pallas import tpu_sc as plsc

NODES = 10000
PAD = 10240
EDGES = 320000
D = 128
NC = 2
NS = 16
NW = NC * NS
EPW = EDGES // NW
K = 80
NCHUNK = EPW // K
RPT = PAD // NS
BR = 1024
RING = 4


def _make_sc_agg():
    mesh = plsc.VectorSubcoreMesh(core_axis_name="c", subcore_axis_name="s")
    out_type = jax.ShapeDtypeStruct((NC, PAD, D), jnp.float32)
    scratch = (
        [pltpu.VMEM((K,), jnp.int32) for _ in range(2 * RING)]
        + [pltpu.VMEM((K, D), jnp.float32) for _ in range(RING)]
        + [pltpu.VMEM_SHARED((PAD, D), jnp.float32)]
        + [pltpu.SemaphoreType.DMA] * (2 * RING)
    )

    def body(x_hbm, ei_hbm, out_hbm, *rest):
        isrc = rest[0:RING]
        idst = rest[RING:2 * RING]
        rows = rest[2 * RING:3 * RING]
        acc = rest[3 * RING]
        gsem = rest[3 * RING + 1:3 * RING + 1 + RING]
        ssem = rest[3 * RING + 1 + RING:]
        c = lax.axis_index("c")
        s = lax.axis_index("s")
        wid = s * NC + c
        zero16 = jnp.zeros((16,), jnp.float32)

        def zero_rows0(i, carry):
            for j in range(D // 16):
                rows[0][i, pl.ds(j * 16, 16)] = zero16
            return carry

        lax.fori_loop(0, K, zero_rows0, 0)
        r0 = s * RPT
        for kk in range(RPT // K):
            pltpu.sync_copy(rows[0], acc.at[pl.ds(r0 + kk * K, K)])
        plsc.subcore_barrier()

        ebase = wid * EPW

        def drain(sem, b):
            pltpu.make_async_copy(x_hbm.at[pl.ds(0, K)], rows[b], sem).wait()

        def load_idx(t, b, sem):
            off = jnp.minimum(ebase + t * K, EDGES - K)
            d0 = pltpu.async_copy(ei_hbm.at[pl.ds(off, K)], isrc[b], sem)
            d1 = pltpu.async_copy(ei_hbm.at[pl.ds(EDGES + off, K)],
                                  idst[b], sem)
            return d0, d1

        def fire_gather(b):
            pltpu.async_copy(x_hbm.at[isrc[b]], rows[b], gsem[b])

        def step(t, b, bn, do_drain):
            if do_drain:
                drain(ssem[bn], bn)
            nxt = load_idx(t + 1, bn, gsem[bn])
            drain_g = pltpu.make_async_copy(x_hbm.at[pl.ds(0, K)], rows[b],
                                            gsem[b])
            drain_g.wait()
            pltpu.async_copy(rows[b], acc.at[idst[b]], ssem[b], add=True)
            for n in nxt:
                n.wait()
            fire_gather(bn)

        for n in load_idx(0, 0, gsem[0]):
            n.wait()
        fire_gather(0)
        step(0, 0, 1, False)
        step(1, 1, 2, False)
        step(2, 2, 3, False)
        step(3, 3, 0, True)

        def quad(i, carry):
            t = 4 + 4 * i
            step(t, 0, 1, True)
            step(t + 1, 1, 2, True)
            step(t + 2, 2, 3, True)
            step(t + 3, 3, 0, True)
            return carry

        lax.fori_loop(0, (NCHUNK - 5) // 4, quad, 0)
        step(NCHUNK - 1, 0, 1, True)
        drain(ssem[2], 2)
        drain(ssem[3], 3)
        drain(ssem[0], 0)
        drain(gsem[1], 1)
        plsc.subcore_barrier()
        pltpu.sync_copy(acc.at[pl.ds(s * RPT, RPT)],
                        out_hbm.at[c, pl.ds(s * RPT, RPT)])

    return functools.partial(
        pl.kernel, mesh=mesh, out_type=out_type,
        scratch_types=tuple(scratch),
        compiler_params=pltpu.CompilerParams(needs_layout_passes=False))(body)


def _make_sc_deg():
    mesh = plsc.VectorSubcoreMesh(core_axis_name="c", subcore_axis_name="s")
    out_type = jax.ShapeDtypeStruct((NW, PAD), jnp.float32)
    scratch = [
        pltpu.VMEM((EPW,), jnp.int32),
        pltpu.VMEM((PAD,), jnp.float32),
    ]

    def body(ei_hbm, degp_hbm, dsts, deg_v):
        c = lax.axis_index("c")
        s = lax.axis_index("s")
        wid = s * NC + c
        zero16 = jnp.zeros((16,), jnp.float32)
        ones16 = jnp.full((16,), 1.0, jnp.float32)

        def zero_deg(i, carry):
            deg_v[pl.ds(i * 16, 16)] = zero16
            return carry

        lax.fori_loop(0, PAD // 16, zero_deg, 0)
        pltpu.sync_copy(ei_hbm.at[pl.ds(EDGES + wid * EPW, EPW)], dsts)

        def grp(r, carry):
            idx = dsts[pl.ds(r * 16, 16)]
            plsc.addupdate_scatter(deg_v, [idx], ones16)
            return carry

        lax.fori_loop(0, EPW // 16, grp, 0)
        pltpu.sync_copy(deg_v, degp_hbm.at[wid])

    return functools.partial(
        pl.kernel, mesh=mesh, out_type=out_type,
        scratch_types=tuple(scratch),
        compiler_params=pltpu.CompilerParams(needs_layout_passes=False))(body)


def _mm_body(x_ref, w_ref, o_ref):
    o_ref[...] = jnp.dot(x_ref[...], w_ref[...],
                         preferred_element_type=jnp.float32)


def _mm(x, w):
    return pl.pallas_call(
        _mm_body,
        grid=(PAD // BR,),
        in_specs=[pl.BlockSpec((BR, D), lambda i: (i, 0)),
                  pl.BlockSpec((D, D), lambda i: (0, 0))],
        out_specs=pl.BlockSpec((BR, D), lambda i: (i, 0)),
        out_shape=jax.ShapeDtypeStruct((NODES, D), jnp.float32),
    )(x, w)


def _combine_body(h_ref, p_ref, degt_ref, ws_ref, b_ref, wn_ref,
                  o1_ref, o2_ref):
    deg = jnp.sum(degt_ref[...], axis=1, keepdims=True)
    inv = 1.0 / jnp.maximum(deg, 1.0)
    agg = (p_ref[0] + p_ref[1]) * inv
    t = jnp.dot(h_ref[...], ws_ref[...],
                preferred_element_type=jnp.float32) + b_ref[...] + agg
    hr = jnp.maximum(t, 0.0)
    o1_ref[...] = hr
    o2_ref[...] = jnp.dot(hr, wn_ref[...],
                          preferred_element_type=jnp.float32)


def _combine(h, p, degt, ws, b, wn):
    return pl.pallas_call(
        _combine_body,
        grid=(PAD // BR,),
        in_specs=[pl.BlockSpec((BR, D), lambda i: (i, 0)),
                  pl.BlockSpec((NC, BR, D), lambda i: (0, i, 0)),
                  pl.BlockSpec((BR, NW), lambda i: (i, 0)),
                  pl.BlockSpec((D, D), lambda i: (0, 0)),
                  pl.BlockSpec((1, D), lambda i: (0, 0)),
                  pl.BlockSpec((D, D), lambda i: (0, 0))],
        out_specs=[pl.BlockSpec((BR, D), lambda i: (i, 0)),
                   pl.BlockSpec((BR, D), lambda i: (i, 0))],
        out_shape=[jax.ShapeDtypeStruct((NODES, D), jnp.float32),
                   jax.ShapeDtypeStruct((NODES, D), jnp.float32)],
    )(h, p, degt, ws, b, wn)


def _final_body(h_ref, p_ref, degt_ref, ws_ref, b_ref, o_ref):
    deg = jnp.sum(degt_ref[...], axis=1, keepdims=True)
    inv = 1.0 / jnp.maximum(deg, 1.0)
    agg = (p_ref[0] + p_ref[1]) * inv
    o_ref[...] = jnp.dot(h_ref[...], ws_ref[...],
                         preferred_element_type=jnp.float32) + b_ref[...] + agg


def _final(h, p, degt, ws, b):
    return pl.pallas_call(
        _final_body,
        grid=(PAD // BR,),
        in_specs=[pl.BlockSpec((BR, D), lambda i: (i, 0)),
                  pl.BlockSpec((NC, BR, D), lambda i: (0, i, 0)),
                  pl.BlockSpec((BR, NW), lambda i: (i, 0)),
                  pl.BlockSpec((D, D), lambda i: (0, 0)),
                  pl.BlockSpec((1, D), lambda i: (0, 0))],
        out_specs=pl.BlockSpec((BR, D), lambda i: (i, 0)),
        out_shape=jax.ShapeDtypeStruct((NODES, D), jnp.float32),
    )(h, p, degt, ws, b)


def kernel(h, edge_index, W_self0, W_neigh0, b0, W_self1, W_neigh1, b1,
           W_self2, W_neigh2, b2):
    b0r = b0.reshape(1, D)
    b1r = b1.reshape(1, D)
    b2r = b2.reshape(1, D)

    sc_agg = _make_sc_agg()
    sc_deg = _make_sc_deg()

    ei = edge_index.reshape(2 * EDGES)
    degp = sc_deg(ei)
    hn0 = _mm(h, W_neigh0)
    p0 = sc_agg(hn0, ei)
    degt = degp.T
    h1, hn1 = _combine(h, p0, degt, W_self0, b0r, W_neigh1)
    p1 = sc_agg(hn1, ei)
    h2, hn2 = _combine(h1, p1, degt, W_self1, b1r, W_neigh2)
    p2 = sc_agg(hn2, ei)
    return _final(h2, p2, degt, W_self2, b2r)


# speedup vs baseline: 1.0695x; 1.0015x over previous
import functools

import jax
import jax.numpy as jnp
from jax import lax
from jax.experimental import pallas as pl
from jax.experimental.pallas import tpu as pltpu
from jax.experimental.pallas import tpu_sc as plsc

NODES = 10000
PAD = 10240
EDGES = 320000
D = 128
NC = 2
NS = 16
NW = NC * NS
EPW = EDGES // NW
K = 80
NCHUNK = EPW // K
RPT = PAD // NS
BR = 1024
RING = 4


def _make_sc_agg():
    mesh = plsc.VectorSubcoreMesh(core_axis_name="c", subcore_axis_name="s")
    out_type = jax.ShapeDtypeStruct((NC, PAD, D), jnp.float32)
    scratch = (
        [pltpu.VMEM((K,), jnp.int32) for _ in range(2 * RING)]
        + [pltpu.VMEM((K, D), jnp.float32) for _ in range(RING)]
        + [pltpu.VMEM_SHARED((PAD, D), jnp.float32)]
        + [pltpu.SemaphoreType.DMA] * (2 * RING)
    )

    def body(x_hbm, ei_hbm, out_hbm, *rest):
        isrc = rest[0:RING]
        idst = rest[RING:2 * RING]
        rows = rest[2 * RING:3 * RING]
        acc = rest[3 * RING]
        gsem = rest[3 * RING + 1:3 * RING + 1 + RING]
        ssem = rest[3 * RING + 1 + RING:]
        c = lax.axis_index("c")
        s = lax.axis_index("s")
        wid = s * NC + c
        zero16 = jnp.zeros((16,), jnp.float32)

        def zero_rows0(i, carry):
            for j in range(D // 16):
                rows[0][i, pl.ds(j * 16, 16)] = zero16
            return carry

        lax.fori_loop(0, K, zero_rows0, 0)
        r0 = s * RPT
        for kk in range(RPT // K):
            pltpu.sync_copy(rows[0], acc.at[pl.ds(r0 + kk * K, K)])
        plsc.subcore_barrier()

        ebase = wid * EPW

        def drain(sem, b):
            pltpu.make_async_copy(x_hbm.at[pl.ds(0, K)], rows[b], sem).wait()

        def load_idx(t, b, sem):
            off = jnp.minimum(ebase + t * K, EDGES - K)
            d0 = pltpu.async_copy(ei_hbm.at[pl.ds(off, K)], isrc[b], sem)
            d1 = pltpu.async_copy(ei_hbm.at[pl.ds(EDGES + off, K)],
                                  idst[b], sem)
            return d0, d1

        def fire_gather(b):
            half = K // 2
            pltpu.async_copy(x_hbm.at[isrc[b].at[pl.ds(0, half)]],
                             rows[b].at[pl.ds(0, half)], gsem[b])
            pltpu.async_copy(x_hbm.at[isrc[b].at[pl.ds(half, half)]],
                             rows[b].at[pl.ds(half, half)], gsem[b])

        def step(t, b, bn, do_drain):
            if do_drain:
                drain(ssem[bn], bn)
            nxt = load_idx(t + 1, bn, gsem[bn])
            drain_g = pltpu.make_async_copy(x_hbm.at[pl.ds(0, K)], rows[b],
                                            gsem[b])
            drain_g.wait()
            pltpu.async_copy(rows[b], acc.at[idst[b]], ssem[b], add=True)
            for n in nxt:
                n.wait()
            fire_gather(bn)

        for n in load_idx(0, 0, gsem[0]):
            n.wait()
        fire_gather(0)
        step(0, 0, 1, False)
        step(1, 1, 2, False)
        step(2, 2, 3, False)
        step(3, 3, 0, True)

        def quad(i, carry):
            t = 4 + 4 * i
            step(t, 0, 1, True)
            step(t + 1, 1, 2, True)
            step(t + 2, 2, 3, True)
            step(t + 3, 3, 0, True)
            return carry

        lax.fori_loop(0, (NCHUNK - 5) // 4, quad, 0)
        step(NCHUNK - 1, 0, 1, True)
        drain(ssem[2], 2)
        drain(ssem[3], 3)
        drain(ssem[0], 0)
        drain(gsem[1], 1)
        plsc.subcore_barrier()
        pltpu.sync_copy(acc.at[pl.ds(s * RPT, RPT)],
                        out_hbm.at[c, pl.ds(s * RPT, RPT)])

    return functools.partial(
        pl.kernel, mesh=mesh, out_type=out_type,
        scratch_types=tuple(scratch),
        compiler_params=pltpu.CompilerParams(needs_layout_passes=False))(body)


def _make_sc_deg():
    mesh = plsc.VectorSubcoreMesh(core_axis_name="c", subcore_axis_name="s")
    out_type = jax.ShapeDtypeStruct((NW, PAD), jnp.float32)
    scratch = [
        pltpu.VMEM((EPW,), jnp.int32),
        pltpu.VMEM((PAD,), jnp.float32),
    ]

    def body(ei_hbm, degp_hbm, dsts, deg_v):
        c = lax.axis_index("c")
        s = lax.axis_index("s")
        wid = s * NC + c
        zero16 = jnp.zeros((16,), jnp.float32)
        ones16 = jnp.full((16,), 1.0, jnp.float32)

        def zero_deg(i, carry):
            deg_v[pl.ds(i * 16, 16)] = zero16
            return carry

        lax.fori_loop(0, PAD // 16, zero_deg, 0)
        pltpu.sync_copy(ei_hbm.at[pl.ds(EDGES + wid * EPW, EPW)], dsts)

        def grp(r, carry):
            idx = dsts[pl.ds(r * 16, 16)]
            plsc.addupdate_scatter(deg_v, [idx], ones16)
            return carry

        lax.fori_loop(0, EPW // 16, grp, 0)
        pltpu.sync_copy(deg_v, degp_hbm.at[wid])

    return functools.partial(
        pl.kernel, mesh=mesh, out_type=out_type,
        scratch_types=tuple(scratch),
        compiler_params=pltpu.CompilerParams(needs_layout_passes=False))(body)


def _mm_body(x_ref, w_ref, o_ref):
    o_ref[...] = jnp.dot(x_ref[...], w_ref[...],
                         preferred_element_type=jnp.float32)


def _mm(x, w):
    return pl.pallas_call(
        _mm_body,
        grid=(PAD // BR,),
        in_specs=[pl.BlockSpec((BR, D), lambda i: (i, 0)),
                  pl.BlockSpec((D, D), lambda i: (0, 0))],
        out_specs=pl.BlockSpec((BR, D), lambda i: (i, 0)),
        out_shape=jax.ShapeDtypeStruct((NODES, D), jnp.float32),
    )(x, w)


def _combine_body(h_ref, p_ref, degt_ref, ws_ref, b_ref, wn_ref,
                  o1_ref, o2_ref):
    deg = jnp.sum(degt_ref[...], axis=1, keepdims=True)
    inv = 1.0 / jnp.maximum(deg, 1.0)
    agg = (p_ref[0] + p_ref[1]) * inv
    t = jnp.dot(h_ref[...], ws_ref[...],
                preferred_element_type=jnp.float32) + b_ref[...] + agg
    hr = jnp.maximum(t, 0.0)
    o1_ref[...] = hr
    o2_ref[...] = jnp.dot(hr, wn_ref[...],
                          preferred_element_type=jnp.float32)


def _combine(h, p, degt, ws, b, wn):
    return pl.pallas_call(
        _combine_body,
        grid=(PAD // BR,),
        in_specs=[pl.BlockSpec((BR, D), lambda i: (i, 0)),
                  pl.BlockSpec((NC, BR, D), lambda i: (0, i, 0)),
                  pl.BlockSpec((BR, NW), lambda i: (i, 0)),
                  pl.BlockSpec((D, D), lambda i: (0, 0)),
                  pl.BlockSpec((1, D), lambda i: (0, 0)),
                  pl.BlockSpec((D, D), lambda i: (0, 0))],
        out_specs=[pl.BlockSpec((BR, D), lambda i: (i, 0)),
                   pl.BlockSpec((BR, D), lambda i: (i, 0))],
        out_shape=[jax.ShapeDtypeStruct((NODES, D), jnp.float32),
                   jax.ShapeDtypeStruct((NODES, D), jnp.float32)],
    )(h, p, degt, ws, b, wn)


def _final_body(h_ref, p_ref, degt_ref, ws_ref, b_ref, o_ref):
    deg = jnp.sum(degt_ref[...], axis=1, keepdims=True)
    inv = 1.0 / jnp.maximum(deg, 1.0)
    agg = (p_ref[0] + p_ref[1]) * inv
    o_ref[...] = jnp.dot(h_ref[...], ws_ref[...],
                         preferred_element_type=jnp.float32) + b_ref[...] + agg


def _final(h, p, degt, ws, b):
    return pl.pallas_call(
        _final_body,
        grid=(PAD // BR,),
        in_specs=[pl.BlockSpec((BR, D), lambda i: (i, 0)),
                  pl.BlockSpec((NC, BR, D), lambda i: (0, i, 0)),
                  pl.BlockSpec((BR, NW), lambda i: (i, 0)),
                  pl.BlockSpec((D, D), lambda i: (0, 0)),
                  pl.BlockSpec((1, D), lambda i: (0, 0))],
        out_specs=pl.BlockSpec((BR, D), lambda i: (i, 0)),
        out_shape=jax.ShapeDtypeStruct((NODES, D), jnp.float32),
    )(h, p, degt, ws, b)


def kernel(h, edge_index, W_self0, W_neigh0, b0, W_self1, W_neigh1, b1,
           W_self2, W_neigh2, b2):
    b0r = b0.reshape(1, D)
    b1r = b1.reshape(1, D)
    b2r = b2.reshape(1, D)

    sc_agg = _make_sc_agg()
    sc_deg = _make_sc_deg()

    ei = edge_index.reshape(2 * EDGES)
    degp = sc_deg(ei)
    hn0 = _mm(h, W_neigh0)
    p0 = sc_agg(hn0, ei)
    degt = degp.T
    h1, hn1 = _combine(h, p0, degt, W_self0, b0r, W_neigh1)
    p1 = sc_agg(hn1, ei)
    h2, hn2 = _combine(h1, p1, degt, W_self1, b1r, W_neigh2)
    p2 = sc_agg(hn2, ei)
    return _final(h2, p2, degt, W_self2, b2r)
